# Initial kernel scaffold; baseline (speedup 1.0000x reference)
import jax
import jax.numpy as jnp
from jax.experimental import pallas as pl


def kernel(x, edge_index, bu_edge_index, batch, W1, b1, W2, b2, W3, b3, W4, b4, Wfc, bfc):
    raise NotImplementedError("write your pallas kernel here")


# speedup vs baseline: 11.0670x; 11.0670x over previous
import functools

import jax
import jax.numpy as jnp
from jax import lax
from jax.experimental import pallas as pl
from jax.experimental.pallas import tpu as pltpu
from jax.experimental.pallas import tpu_sc as plsc

_NC = 2
_NS = 16
_K = 80
_G = 64
_HW = 128


def _sc_mesh():
    return plsc.VectorSubcoreMesh(core_axis_name="c", subcore_axis_name="s")


@functools.lru_cache(maxsize=None)
def _make_deg_kernel(E, N):
    assert E % (_NS * _K) == 0 and N % _NS == 0
    rows_per_tile = N // _NS
    chunks = E // (_NS * _K)
    edges_per_tile = chunks * _K

    @functools.partial(
        pl.kernel,
        out_type=[jax.ShapeDtypeStruct((N, _HW), jnp.float32)] * 2,
        mesh=_sc_mesh(),
        scratch_types=[
            pltpu.VMEM((_K,), jnp.int32),
            pltpu.VMEM((_K, _HW), jnp.float32),
            pltpu.VMEM_SHARED((N, _HW), jnp.float32),
            pltpu.SemaphoreType.DMA,
        ],
    )
    def deg_kernel(dst_td, dst_bu, zeros_nw, ones_kw, out_td, out_bu,
                   idx_v, ones_v, hist_s, sem):
        c = lax.axis_index("c")
        s = lax.axis_index("s")
        r0 = s * rows_per_tile
        pltpu.sync_copy(zeros_nw, hist_s.at[pl.ds(r0, rows_per_tile)])
        pltpu.sync_copy(ones_kw, ones_v)
        plsc.subcore_barrier()

        def run(dst_hbm):
            base0 = s * edges_per_tile

            def step(i, carry):
                base = pl.multiple_of(base0 + i * _K, 8)
                pltpu.sync_copy(dst_hbm.at[pl.ds(base, _K)], idx_v)
                pltpu.sync_copy(ones_v, hist_s.at[idx_v], add=True)
                return carry

            lax.fori_loop(0, chunks, step, 0)

        @pl.when(c == 0)
        def _():
            run(dst_td)

        @pl.when(c == 1)
        def _():
            run(dst_bu)

        plsc.subcore_barrier()

        @pl.when(c == 0)
        def _():
            pltpu.sync_copy(hist_s.at[pl.ds(r0, rows_per_tile)],
                            out_td.at[pl.ds(r0, rows_per_tile)])

        @pl.when(c == 1)
        def _():
            pltpu.sync_copy(hist_s.at[pl.ds(r0, rows_per_tile)],
                            out_bu.at[pl.ds(r0, rows_per_tile)])

    return deg_kernel


@functools.lru_cache(maxsize=None)
def _make_agg_kernel(E, N, D):
    assert E % (_NS * _K) == 0 and N % _NS == 0
    rows_per_tile = N // _NS
    chunks = E // (_NS * _K)
    edges_per_tile = chunks * _K

    @functools.partial(
        pl.kernel,
        out_type=[jax.ShapeDtypeStruct((N, D), jnp.float32)] * 2,
        mesh=_sc_mesh(),
        scratch_types=[
            pltpu.VMEM((_K,), jnp.int32),
            pltpu.VMEM((_K,), jnp.int32),
            pltpu.VMEM((_K, D), jnp.float32),
            pltpu.VMEM_SHARED((N, D), jnp.float32),
            pltpu.SemaphoreType.DMA,
        ],
    )
    def agg_kernel(y_td, src_td, dst_td, y_bu, src_bu, dst_bu, zeros_nd,
                   z_td, z_bu, sidx_v, didx_v, rows_v, acc_s, sem):
        c = lax.axis_index("c")
        s = lax.axis_index("s")
        r0 = s * rows_per_tile
        pltpu.sync_copy(zeros_nd, acc_s.at[pl.ds(r0, rows_per_tile)])
        plsc.subcore_barrier()

        def run(y_hbm, src_hbm, dst_hbm):
            base0 = s * edges_per_tile

            def step(i, carry):
                base = pl.multiple_of(base0 + i * _K, 8)
                pltpu.sync_copy(src_hbm.at[pl.ds(base, _K)], sidx_v)
                pltpu.async_copy(y_hbm.at[sidx_v], rows_v, sem).wait()
                pltpu.sync_copy(dst_hbm.at[pl.ds(base, _K)], didx_v)
                pltpu.sync_copy(rows_v, acc_s.at[didx_v], add=True)
                return carry

            lax.fori_loop(0, chunks, step, 0)

        @pl.when(c == 0)
        def _():
            run(y_td, src_td, dst_td)

        @pl.when(c == 1)
        def _():
            run(y_bu, src_bu, dst_bu)

        plsc.subcore_barrier()

        @pl.when(c == 0)
        def _():
            pltpu.sync_copy(acc_s.at[pl.ds(r0, rows_per_tile)],
                            z_td.at[pl.ds(r0, rows_per_tile)])

        @pl.when(c == 1)
        def _():
            pltpu.sync_copy(acc_s.at[pl.ds(r0, rows_per_tile)],
                            z_bu.at[pl.ds(r0, rows_per_tile)])

    return agg_kernel


def _dot(a, b):
    return jnp.dot(a, b, preferred_element_type=jnp.float32,
                   precision=lax.Precision.HIGHEST)


def _elu(v):
    return jnp.where(v > 0.0, v, jnp.exp(jnp.minimum(v, 0.0)) - 1.0)


def _tc1_body(x_ref, degtd_ref, degbu_ref, w1_ref, w3_ref,
              y1_ref, y3_ref, itd_ref, ibu_ref):
    itd = lax.rsqrt(degtd_ref[:, 0:1] + 1.0)
    ibu = lax.rsqrt(degbu_ref[:, 0:1] + 1.0)
    xv = x_ref[...]
    y1_ref[...] = itd * _dot(xv, w1_ref[...])
    y3_ref[...] = ibu * _dot(xv, w3_ref[...])
    itd_ref[...] = itd
    ibu_ref[...] = ibu


def _tc2_body(z1_ref, y1_ref, itd_ref, b1_ref, w2_ref,
              z3_ref, y3_ref, ibu_ref, b3_ref, w4_ref,
              y2_ref, y4_ref):
    itd = itd_ref[...]
    ibu = ibu_ref[...]
    h1 = _elu(itd * (z1_ref[...] + y1_ref[...]) + b1_ref[...])
    h3 = _elu(ibu * (z3_ref[...] + y3_ref[...]) + b3_ref[...])
    y2_ref[...] = itd * _dot(h1, w2_ref[...])
    y4_ref[...] = ibu * _dot(h3, w4_ref[...])


def _tc3_body(nblocks, z2_ref, y2_ref, itd_ref, b2_ref,
              z4_ref, y4_ref, ibu_ref, b4_ref,
              batch_ref, wfc_ref, bfc_ref, out_ref,
              acc_td, acc_bu, acc_cnt):
    i = pl.program_id(0)

    @pl.when(i == 0)
    def _():
        acc_td[...] = jnp.zeros_like(acc_td)
        acc_bu[...] = jnp.zeros_like(acc_bu)
        acc_cnt[...] = jnp.zeros_like(acc_cnt)

    h2 = _elu(itd_ref[...] * (z2_ref[...] + y2_ref[...]) + b2_ref[...])
    h4 = _elu(ibu_ref[...] * (z4_ref[...] + y4_ref[...]) + b4_ref[...])
    r = h2.shape[0]
    b = batch_ref[0]
    pt = (lax.broadcasted_iota(jnp.int32, (_G, r), 0) == b)
    pt = pt.astype(jnp.float32)
    acc_td[...] += _dot(pt, h2)
    acc_bu[...] += _dot(pt, h4)
    acc_cnt[...] += jnp.sum(pt, axis=1, keepdims=True)

    @pl.when(i == nblocks - 1)
    def _():
        inv = 1.0 / jnp.maximum(acc_cnt[...], 1.0)
        z = jnp.concatenate([acc_td[...] * inv, acc_bu[...] * inv], axis=1)
        logits = _dot(z, wfc_ref[...]) + bfc_ref[...]
        mx = jnp.max(logits, axis=1, keepdims=True)
        lse = jnp.log(jnp.sum(jnp.exp(logits - mx), axis=1, keepdims=True))
        out_ref[...] = (logits - mx) - lse


def kernel(x, edge_index, bu_edge_index, batch,
           W1, b1, W2, b2, W3, b3, W4, b4, Wfc, bfc):
    N0, D = x.shape
    H = W1.shape[1]
    C = Wfc.shape[1]
    E = edge_index.shape[1]
    f32 = jnp.float32

    N = ((N0 + 127) // 128) * 128
    if N != N0:
        x = jnp.concatenate([x, jnp.zeros((N - N0, D), f32)], axis=0)
        batch = jnp.concatenate(
            [batch, jnp.full((N - N0,), _G, batch.dtype)], axis=0)

    src_td, dst_td = edge_index[0], edge_index[1]
    src_bu, dst_bu = bu_edge_index[0], bu_edge_index[1]

    zeros_hw = jnp.zeros((N // _NS, _HW), f32)
    ones_kw = jnp.ones((_K, _HW), f32)
    zeros_nd = jnp.zeros((N // _NS, D), f32)

    deg_td, deg_bu = _make_deg_kernel(E, N)(dst_td, dst_bu, zeros_hw, ones_kw)

    NB = 8
    R = N // NB
    row = pl.BlockSpec((R, H), lambda i: (i, 0))
    row1 = pl.BlockSpec((R, 1), lambda i: (i, 0))
    roww = pl.BlockSpec((R, _HW), lambda i: (i, 0))
    wspec = pl.BlockSpec((H, H), lambda i: (0, 0))
    bspec = pl.BlockSpec((1, H), lambda i: (0, 0))

    y1, y3, itd, ibu = pl.pallas_call(
        _tc1_body,
        grid=(NB,),
        in_specs=[row, roww, roww, wspec, wspec],
        out_specs=[row, row, row1, row1],
        out_shape=[
            jax.ShapeDtypeStruct((N, H), f32),
            jax.ShapeDtypeStruct((N, H), f32),
            jax.ShapeDtypeStruct((N, 1), f32),
            jax.ShapeDtypeStruct((N, 1), f32),
        ],
    )(x, deg_td, deg_bu, W1, W3)

    agg = _make_agg_kernel(E, N, H)
    z1, z3 = agg(y1, src_td, dst_td, y3, src_bu, dst_bu, zeros_nd)

    y2, y4 = pl.pallas_call(
        _tc2_body,
        grid=(NB,),
        in_specs=[row, row, row1, bspec, wspec,
                  row, row, row1, bspec, wspec],
        out_specs=[row, row],
        out_shape=[
            jax.ShapeDtypeStruct((N, H), f32),
            jax.ShapeDtypeStruct((N, H), f32),
        ],
    )(z1, y1, itd, b1.reshape(1, H), W2,
      z3, y3, ibu, b3.reshape(1, H), W4)

    z2, z4 = agg(y2, src_td, dst_td, y4, src_bu, dst_bu, zeros_nd)

    out = pl.pallas_call(
        functools.partial(_tc3_body, NB),
        grid=(NB,),
        in_specs=[row, row, row1, bspec,
                  row, row, row1, bspec,
                  pl.BlockSpec((1, 1, R), lambda i: (i, 0, 0)),
                  pl.BlockSpec((2 * H, C), lambda i: (0, 0)),
                  pl.BlockSpec((1, C), lambda i: (0, 0))],
        out_specs=pl.BlockSpec((_G, C), lambda i: (0, 0)),
        out_shape=jax.ShapeDtypeStruct((_G, C), f32),
        scratch_shapes=[
            pltpu.VMEM((_G, H), f32),
            pltpu.VMEM((_G, H), f32),
            pltpu.VMEM((_G, 1), f32),
        ],
    )(z2, y2, itd, b2.reshape(1, H),
      z4, y4, ibu, b4.reshape(1, H),
      batch.reshape(NB, 1, R), Wfc, bfc.reshape(1, C))

    return out


# speedup vs baseline: 11.3983x; 1.0299x over previous
import functools

import jax
import jax.numpy as jnp
from jax import lax
from jax.experimental import pallas as pl
from jax.experimental.pallas import tpu as pltpu
from jax.experimental.pallas import tpu_sc as plsc

_NC = 2
_NS = 16
_K = 128
_G = 64


def _sc_mesh():
    return plsc.VectorSubcoreMesh(core_axis_name="c", subcore_axis_name="s")


@functools.lru_cache(maxsize=None)
def _make_deg_kernel(EP, N):
    assert EP % (_NS * _K) == 0 and N % _NS == 0
    chunks = EP // (_NS * _K)
    assert chunks % 8 == 0
    rows_per_tile = N // _NS
    _FK = 8

    @functools.partial(
        pl.kernel,
        out_type=[jax.ShapeDtypeStruct((N,), jnp.float32)] * 2,
        mesh=_sc_mesh(),
        scratch_types=[
            pltpu.VMEM((chunks, _K), jnp.int32),
            pltpu.VMEM((_K,), jnp.float32),
            pltpu.VMEM((N // _NS,), jnp.float32),
            pltpu.VMEM_SHARED((N,), jnp.float32),
            pltpu.SemaphoreType.DMA,
        ],
    )
    def deg_kernel(dst_td, dst_bu, zeros_n, ones_k, out_td, out_bu,
                   didx_v, ones_v, stage_v, hist_s, sem):
        c = lax.axis_index("c")
        s = lax.axis_index("s")
        r0 = s * rows_per_tile
        pltpu.sync_copy(zeros_n, stage_v)
        pltpu.sync_copy(stage_v, hist_s.at[pl.ds(r0, rows_per_tile)])
        pltpu.sync_copy(ones_k, ones_v)
        plsc.subcore_barrier()

        def run(dst2d):
            pltpu.sync_copy(dst2d.at[pl.ds(s * chunks, chunks)], didx_v)

            def group(g, carry):
                for q in range(_FK):
                    pltpu.async_copy(
                        ones_v, hist_s.at[didx_v.at[g * _FK + q]], sem,
                        add=True)
                for q in range(_FK):
                    pltpu.make_async_copy(
                        ones_v, hist_s.at[didx_v.at[g * _FK + q]], sem).wait()
                return carry

            lax.fori_loop(0, chunks // _FK, group, 0)

        @pl.when(c == 0)
        def _():
            run(dst_td)

        @pl.when(c == 1)
        def _():
            run(dst_bu)

        plsc.subcore_barrier()

        pltpu.sync_copy(hist_s.at[pl.ds(r0, rows_per_tile)], stage_v)

        @pl.when(c == 0)
        def _():
            pltpu.sync_copy(stage_v, out_td.at[pl.ds(r0, rows_per_tile)])

        @pl.when(c == 1)
        def _():
            pltpu.sync_copy(stage_v, out_bu.at[pl.ds(r0, rows_per_tile)])

    return deg_kernel


@functools.lru_cache(maxsize=None)
def _make_agg_kernel(EP, N, D):
    assert EP % (_NS * _K) == 0 and N % _NS == 0
    chunks = EP // (_NS * _K)
    assert chunks % 8 == 0
    rows_per_tile = N // _NS

    @functools.partial(
        pl.kernel,
        out_type=[jax.ShapeDtypeStruct((N, D), jnp.float32)] * 2,
        mesh=_sc_mesh(),
        scratch_types=[
            pltpu.VMEM((_K,), jnp.int32),
            pltpu.VMEM((_K,), jnp.int32),
            pltpu.VMEM((_K,), jnp.int32),
            pltpu.VMEM((_K,), jnp.int32),
            pltpu.VMEM((_K, D), jnp.float32),
            pltpu.VMEM((_K, D), jnp.float32),
            pltpu.VMEM_SHARED((N, D), jnp.float32),
            pltpu.SemaphoreType.DMA,
            pltpu.SemaphoreType.DMA,
            pltpu.SemaphoreType.DMA,
            pltpu.SemaphoreType.DMA,
        ],
    )
    def agg_kernel(y_td, src_td, dst_td, y_bu, src_bu, dst_bu, zeros_nd,
                   z_td, z_bu, sidx_a, didx_a, sidx_b, didx_b,
                   buf_a, buf_b, acc_s, gsem_a, gsem_b, ssem_a, ssem_b):
        c = lax.axis_index("c")
        s = lax.axis_index("s")
        r0 = s * rows_per_tile
        pltpu.sync_copy(zeros_nd, acc_s.at[pl.ds(r0, rows_per_tile)])
        plsc.subcore_barrier()

        def run(y_hbm, src_hbm, dst_hbm):
            base0 = s * chunks * _K

            def iload(i, sidx, didx):
                base = pl.multiple_of(base0 + i * _K, 8)
                pltpu.sync_copy(src_hbm.at[pl.ds(base, _K)], sidx)
                pltpu.sync_copy(dst_hbm.at[pl.ds(base, _K)], didx)

            def gstart(sidx, buf, sem):
                pltpu.async_copy(y_hbm.at[sidx], buf, sem)

            def gwait(sidx, buf, sem):
                pltpu.make_async_copy(y_hbm.at[sidx], buf, sem).wait()

            def sstart(didx, buf, sem):
                pltpu.async_copy(buf, acc_s.at[didx], sem, add=True)

            def swait(didx, buf, sem):
                pltpu.make_async_copy(buf, acc_s.at[didx], sem).wait()

            iload(0, sidx_a, didx_a)
            gstart(sidx_a, buf_a, gsem_a)
            iload(1, sidx_b, didx_b)
            gstart(sidx_b, buf_b, gsem_b)

            def pair(g, carry):
                i0 = 2 * g
                gwait(sidx_a, buf_a, gsem_a)
                sstart(didx_a, buf_a, ssem_a)
                gwait(sidx_b, buf_b, gsem_b)
                sstart(didx_b, buf_b, ssem_b)
                swait(didx_a, buf_a, ssem_a)

                @pl.when(i0 + 2 < chunks)
                def _():
                    iload(i0 + 2, sidx_a, didx_a)
                    gstart(sidx_a, buf_a, gsem_a)

                swait(didx_b, buf_b, ssem_b)

                @pl.when(i0 + 3 < chunks)
                def _():
                    iload(i0 + 3, sidx_b, didx_b)
                    gstart(sidx_b, buf_b, gsem_b)

                return carry

            lax.fori_loop(0, chunks // 2, pair, 0)

        @pl.when(c == 0)
        def _():
            run(y_td, src_td, dst_td)

        @pl.when(c == 1)
        def _():
            run(y_bu, src_bu, dst_bu)

        plsc.subcore_barrier()

        @pl.when(c == 0)
        def _():
            pltpu.sync_copy(acc_s.at[pl.ds(r0, rows_per_tile)],
                            z_td.at[pl.ds(r0, rows_per_tile)])

        @pl.when(c == 1)
        def _():
            pltpu.sync_copy(acc_s.at[pl.ds(r0, rows_per_tile)],
                            z_bu.at[pl.ds(r0, rows_per_tile)])

    return agg_kernel


def _dot(a, b):
    return jnp.dot(a, b, preferred_element_type=jnp.float32,
                   precision=lax.Precision.HIGHEST)


def _elu(v):
    return jnp.where(v > 0.0, v, jnp.exp(jnp.minimum(v, 0.0)) - 1.0)


def _tc1_body(x_ref, degtd_ref, degbu_ref, w1_ref, w3_ref,
              y1_ref, y3_ref, itd_ref, ibu_ref):
    itd = lax.rsqrt(degtd_ref[...] + 1.0)
    ibu = lax.rsqrt(degbu_ref[...] + 1.0)
    xv = x_ref[...]
    y1_ref[...] = itd * _dot(xv, w1_ref[...])
    y3_ref[...] = ibu * _dot(xv, w3_ref[...])
    itd_ref[...] = itd
    ibu_ref[...] = ibu


def _tc2_body(z1_ref, y1_ref, itd_ref, b1_ref, w2_ref,
              z3_ref, y3_ref, ibu_ref, b3_ref, w4_ref,
              y2_ref, y4_ref):
    itd = itd_ref[...]
    ibu = ibu_ref[...]
    h1 = _elu(itd * (z1_ref[...] + y1_ref[...]) + b1_ref[...])
    h3 = _elu(ibu * (z3_ref[...] + y3_ref[...]) + b3_ref[...])
    y2_ref[...] = itd * _dot(h1, w2_ref[...])
    y4_ref[...] = ibu * _dot(h3, w4_ref[...])


def _tc3_body(nblocks, z2_ref, y2_ref, itd_ref, b2_ref,
              z4_ref, y4_ref, ibu_ref, b4_ref,
              batch_ref, wfc_ref, bfc_ref, out_ref,
              acc_td, acc_bu, acc_cnt):
    i = pl.program_id(0)

    @pl.when(i == 0)
    def _():
        acc_td[...] = jnp.zeros_like(acc_td)
        acc_bu[...] = jnp.zeros_like(acc_bu)
        acc_cnt[...] = jnp.zeros_like(acc_cnt)

    h2 = _elu(itd_ref[...] * (z2_ref[...] + y2_ref[...]) + b2_ref[...])
    h4 = _elu(ibu_ref[...] * (z4_ref[...] + y4_ref[...]) + b4_ref[...])
    r = h2.shape[0]
    b = batch_ref[0]
    pt = (lax.broadcasted_iota(jnp.int32, (_G, r), 0) == b)
    pt = pt.astype(jnp.float32)
    acc_td[...] += _dot(pt, h2)
    acc_bu[...] += _dot(pt, h4)
    acc_cnt[...] += jnp.sum(pt, axis=1, keepdims=True)

    @pl.when(i == nblocks - 1)
    def _():
        inv = 1.0 / jnp.maximum(acc_cnt[...], 1.0)
        z = jnp.concatenate([acc_td[...] * inv, acc_bu[...] * inv], axis=1)
        logits = _dot(z, wfc_ref[...]) + bfc_ref[...]
        mx = jnp.max(logits, axis=1, keepdims=True)
        lse = jnp.log(jnp.sum(jnp.exp(logits - mx), axis=1, keepdims=True))
        out_ref[...] = (logits - mx) - lse


def kernel(x, edge_index, bu_edge_index, batch,
           W1, b1, W2, b2, W3, b3, W4, b4, Wfc, bfc):
    N0, D = x.shape
    H = W1.shape[1]
    C = Wfc.shape[1]
    E0 = edge_index.shape[1]
    f32 = jnp.float32

    N = ((N0 + 127) // 128) * 128
    if N != N0:
        x = jnp.concatenate([x, jnp.zeros((N - N0, D), f32)], axis=0)
        batch = jnp.concatenate(
            [batch, jnp.full((N - N0,), _G, batch.dtype)], axis=0)

    EPQ = _NS * _K * 8
    E = ((E0 + EPQ - 1) // EPQ) * EPQ
    assert N > N0 or E == E0

    def prep(ei):
        src, dst = ei[0], ei[1]
        if E != E0:
            pad = jnp.full((E - E0,), N0, jnp.int32)
            src = jnp.concatenate([src, pad])
            dst = jnp.concatenate([dst, pad])
        return src, dst, dst.reshape(E // _K, _K)

    src_td, dst_td, dst2d_td = prep(edge_index)
    src_bu, dst_bu, dst2d_bu = prep(bu_edge_index)

    zeros_n = jnp.zeros((N // _NS,), f32)
    ones_k = jnp.ones((_K,), f32)
    zeros_nd = jnp.zeros((N // _NS, D), f32)

    deg_td, deg_bu = _make_deg_kernel(E, N)(dst2d_td, dst2d_bu,
                                            zeros_n, ones_k)
    deg_td = deg_td.reshape(N, 1)
    deg_bu = deg_bu.reshape(N, 1)

    NB = 8
    R = N // NB
    row = pl.BlockSpec((R, H), lambda i: (i, 0))
    row1 = pl.BlockSpec((R, 1), lambda i: (i, 0))
    wspec = pl.BlockSpec((H, H), lambda i: (0, 0))
    bspec = pl.BlockSpec((1, H), lambda i: (0, 0))

    y1, y3, itd, ibu = pl.pallas_call(
        _tc1_body,
        grid=(NB,),
        in_specs=[row, row1, row1, wspec, wspec],
        out_specs=[row, row, row1, row1],
        out_shape=[
            jax.ShapeDtypeStruct((N, H), f32),
            jax.ShapeDtypeStruct((N, H), f32),
            jax.ShapeDtypeStruct((N, 1), f32),
            jax.ShapeDtypeStruct((N, 1), f32),
        ],
    )(x, deg_td, deg_bu, W1, W3)

    agg = _make_agg_kernel(E, N, H)
    z1, z3 = agg(y1, src_td, dst_td, y3, src_bu, dst_bu, zeros_nd)

    y2, y4 = pl.pallas_call(
        _tc2_body,
        grid=(NB,),
        in_specs=[row, row, row1, bspec, wspec,
                  row, row, row1, bspec, wspec],
        out_specs=[row, row],
        out_shape=[
            jax.ShapeDtypeStruct((N, H), f32),
            jax.ShapeDtypeStruct((N, H), f32),
        ],
    )(z1, y1, itd, b1.reshape(1, H), W2,
      z3, y3, ibu, b3.reshape(1, H), W4)

    z2, z4 = agg(y2, src_td, dst_td, y4, src_bu, dst_bu, zeros_nd)

    out = pl.pallas_call(
        functools.partial(_tc3_body, NB),
        grid=(NB,),
        in_specs=[row, row, row1, bspec,
                  row, row, row1, bspec,
                  pl.BlockSpec((1, 1, R), lambda i: (i, 0, 0)),
                  pl.BlockSpec((2 * H, C), lambda i: (0, 0)),
                  pl.BlockSpec((1, C), lambda i: (0, 0))],
        out_specs=pl.BlockSpec((_G, C), lambda i: (0, 0)),
        out_shape=jax.ShapeDtypeStruct((_G, C), f32),
        scratch_shapes=[
            pltpu.VMEM((_G, H), f32),
            pltpu.VMEM((_G, H), f32),
            pltpu.VMEM((_G, 1), f32),
        ],
    )(z2, y2, itd, b2.reshape(1, H),
      z4, y4, ibu, b4.reshape(1, H),
      batch.reshape(NB, 1, R), Wfc, bfc.reshape(1, C))

    return out


# speedup vs baseline: 12.6760x; 1.1121x over previous
import functools

import jax
import jax.numpy as jnp
from jax import lax
from jax.experimental import pallas as pl
from jax.experimental.pallas import tpu as pltpu
from jax.experimental.pallas import tpu_sc as plsc

_NC = 2
_NS = 16
_K = 128
_G = 64


def _sc_mesh():
    return plsc.VectorSubcoreMesh(core_axis_name="c", subcore_axis_name="s")


@functools.lru_cache(maxsize=None)
def _make_deg_kernel(EP, N):
    assert EP % (_NS * _K) == 0 and N % _NS == 0
    chunks = EP // (_NS * _K)
    assert chunks % 8 == 0
    rows_per_tile = N // _NS
    _FK = 8

    @functools.partial(
        pl.kernel,
        out_type=[jax.ShapeDtypeStruct((N,), jnp.float32)] * 2,
        mesh=_sc_mesh(),
        scratch_types=[
            pltpu.VMEM((chunks, _K), jnp.int32),
            pltpu.VMEM((_K,), jnp.float32),
            pltpu.VMEM((N // _NS,), jnp.float32),
            pltpu.VMEM_SHARED((N,), jnp.float32),
            pltpu.SemaphoreType.DMA,
        ],
    )
    def deg_kernel(dst_td, dst_bu, zeros_n, ones_k, out_td, out_bu,
                   didx_v, ones_v, stage_v, hist_s, sem):
        c = lax.axis_index("c")
        s = lax.axis_index("s")
        r0 = s * rows_per_tile
        pltpu.sync_copy(zeros_n, stage_v)
        pltpu.sync_copy(stage_v, hist_s.at[pl.ds(r0, rows_per_tile)])
        pltpu.sync_copy(ones_k, ones_v)
        plsc.subcore_barrier()

        def run(dst2d):
            pltpu.sync_copy(dst2d.at[pl.ds(s * chunks, chunks)], didx_v)

            def group(g, carry):
                for q in range(_FK):
                    pltpu.async_copy(
                        ones_v, hist_s.at[didx_v.at[g * _FK + q]], sem,
                        add=True)
                for q in range(_FK):
                    pltpu.make_async_copy(
                        ones_v, hist_s.at[didx_v.at[g * _FK + q]], sem).wait()
                return carry

            lax.fori_loop(0, chunks // _FK, group, 0)

        @pl.when(c == 0)
        def _():
            run(dst_td)

        @pl.when(c == 1)
        def _():
            run(dst_bu)

        plsc.subcore_barrier()

        pltpu.sync_copy(hist_s.at[pl.ds(r0, rows_per_tile)], stage_v)

        @pl.when(c == 0)
        def _():
            pltpu.sync_copy(stage_v, out_td.at[pl.ds(r0, rows_per_tile)])

        @pl.when(c == 1)
        def _():
            pltpu.sync_copy(stage_v, out_bu.at[pl.ds(r0, rows_per_tile)])

    return deg_kernel


@functools.lru_cache(maxsize=None)
def _make_agg_kernel(EP, N, D):
    assert EP % (_NS * _K) == 0 and N % _NS == 0
    chunks = EP // (_NS * _K)
    BODY = 16
    assert chunks % BODY == 0
    nbody = chunks // BODY
    rows_per_tile = N // _NS

    @functools.partial(
        pl.kernel,
        out_type=[jax.ShapeDtypeStruct((N, D), jnp.float32)] * 2,
        mesh=_sc_mesh(),
        scratch_types=[
            pltpu.VMEM((8, _K), jnp.int32),
            pltpu.VMEM((8, _K), jnp.int32),
            pltpu.VMEM((8, _K), jnp.int32),
            pltpu.VMEM((8, _K), jnp.int32),
            pltpu.VMEM((_K, D), jnp.float32),
            pltpu.VMEM((_K, D), jnp.float32),
            pltpu.VMEM_SHARED((N, D), jnp.float32),
            pltpu.SemaphoreType.DMA,
            pltpu.SemaphoreType.DMA,
            pltpu.SemaphoreType.DMA,
            pltpu.SemaphoreType.DMA,
            pltpu.SemaphoreType.DMA,
            pltpu.SemaphoreType.DMA,
        ],
    )
    def agg_kernel(y_td, src_td, dst_td, y_bu, src_bu, dst_bu, zeros_nd,
                   z_td, z_bu, sidx0, didx0, sidx1, didx1,
                   buf_a, buf_b, acc_s,
                   gsem_a, gsem_b, ssem_a, ssem_b, isem0, isem1):
        c = lax.axis_index("c")
        s = lax.axis_index("s")
        r0 = s * rows_per_tile
        pltpu.sync_copy(zeros_nd, acc_s.at[pl.ds(r0, rows_per_tile)])
        plsc.subcore_barrier()

        sidx = (sidx0, sidx1)
        didx = (didx0, didx1)
        isem = (isem0, isem1)
        bufs = (buf_a, buf_b)
        gsem = (gsem_a, gsem_b)
        ssem = (ssem_a, ssem_b)

        def run(y_hbm, src2d, dst2d):
            row_base = s * chunks

            def iload_sync(p, crow):
                r = pl.multiple_of(row_base + crow, 8)
                pltpu.sync_copy(src2d.at[pl.ds(r, 8)], sidx[p])
                pltpu.sync_copy(dst2d.at[pl.ds(r, 8)], didx[p])

            def iload(p, crow):
                r = pl.multiple_of(row_base + crow, 8)
                pltpu.async_copy(src2d.at[pl.ds(r, 8)], sidx[p], isem[p])
                pltpu.async_copy(dst2d.at[pl.ds(r, 8)], didx[p], isem[p])

            def iwait(p):
                pltpu.make_async_copy(src2d.at[pl.ds(row_base, 8)],
                                      sidx[p], isem[p]).wait()
                pltpu.make_async_copy(dst2d.at[pl.ds(row_base, 8)],
                                      didx[p], isem[p]).wait()

            def gstart(q, p, row):
                pltpu.async_copy(y_hbm.at[sidx[p].at[row]], bufs[q], gsem[q])

            def gwait(q):
                pltpu.make_async_copy(y_hbm.at[sidx[0].at[0]],
                                      bufs[q], gsem[q]).wait()

            def sstart(q, p, row):
                pltpu.async_copy(bufs[q], acc_s.at[didx[p].at[row]],
                                 ssem[q], add=True)

            def swait(q):
                pltpu.make_async_copy(bufs[q], acc_s.at[didx[0].at[0]],
                                      ssem[q]).wait()

            iload_sync(0, 0)
            gstart(0, 0, 0)

            def body(t, carry):
                for q in range(BODY):
                    slot = q % 2
                    p = 0 if q < 8 else 1
                    row = q % 8
                    if q == 0:
                        @pl.when(t > 0)
                        def _():
                            swait(1 - slot)
                    else:
                        swait(1 - slot)
                    if q == 1:
                        iload(1, t * BODY + 8)
                    if q == 7:
                        iwait(1)
                    if q == 9:
                        @pl.when(t + 1 < nbody)
                        def _():
                            iload(0, (t + 1) * BODY)
                    if q == 15:
                        @pl.when(t + 1 < nbody)
                        def _():
                            iwait(0)
                            gstart(1 - slot, 0, 0)
                    else:
                        gstart(1 - slot, 0 if q + 1 < 8 else 1, (q + 1) % 8)
                    gwait(slot)
                    sstart(slot, p, row)
                return carry

            lax.fori_loop(0, nbody, body, 0)
            swait(1)

        @pl.when(c == 0)
        def _():
            run(y_td, src_td, dst_td)

        @pl.when(c == 1)
        def _():
            run(y_bu, src_bu, dst_bu)

        plsc.subcore_barrier()

        @pl.when(c == 0)
        def _():
            pltpu.sync_copy(acc_s.at[pl.ds(r0, rows_per_tile)],
                            z_td.at[pl.ds(r0, rows_per_tile)])

        @pl.when(c == 1)
        def _():
            pltpu.sync_copy(acc_s.at[pl.ds(r0, rows_per_tile)],
                            z_bu.at[pl.ds(r0, rows_per_tile)])

    return agg_kernel


def _dot(a, b):
    return jnp.dot(a, b, preferred_element_type=jnp.float32,
                   precision=lax.Precision.HIGHEST)


def _elu(v):
    return jnp.where(v > 0.0, v, jnp.exp(jnp.minimum(v, 0.0)) - 1.0)


def _tc1_body(x_ref, degtd_ref, degbu_ref, w1_ref, w3_ref,
              y1_ref, y3_ref, itd_ref, ibu_ref):
    itd = lax.rsqrt(degtd_ref[...] + 1.0)
    ibu = lax.rsqrt(degbu_ref[...] + 1.0)
    xv = x_ref[...]
    y1_ref[...] = itd * _dot(xv, w1_ref[...])
    y3_ref[...] = ibu * _dot(xv, w3_ref[...])
    itd_ref[...] = itd
    ibu_ref[...] = ibu


def _tc2_body(z1_ref, y1_ref, itd_ref, b1_ref, w2_ref,
              z3_ref, y3_ref, ibu_ref, b3_ref, w4_ref,
              y2_ref, y4_ref):
    itd = itd_ref[...]
    ibu = ibu_ref[...]
    h1 = _elu(itd * (z1_ref[...] + y1_ref[...]) + b1_ref[...])
    h3 = _elu(ibu * (z3_ref[...] + y3_ref[...]) + b3_ref[...])
    y2_ref[...] = itd * _dot(h1, w2_ref[...])
    y4_ref[...] = ibu * _dot(h3, w4_ref[...])


def _tc3_body(nblocks, z2_ref, y2_ref, itd_ref, b2_ref,
              z4_ref, y4_ref, ibu_ref, b4_ref,
              batch_ref, wfc_ref, bfc_ref, out_ref,
              acc_td, acc_bu, acc_cnt):
    i = pl.program_id(0)

    @pl.when(i == 0)
    def _():
        acc_td[...] = jnp.zeros_like(acc_td)
        acc_bu[...] = jnp.zeros_like(acc_bu)
        acc_cnt[...] = jnp.zeros_like(acc_cnt)

    h2 = _elu(itd_ref[...] * (z2_ref[...] + y2_ref[...]) + b2_ref[...])
    h4 = _elu(ibu_ref[...] * (z4_ref[...] + y4_ref[...]) + b4_ref[...])
    r = h2.shape[0]
    b = batch_ref[0]
    pt = (lax.broadcasted_iota(jnp.int32, (_G, r), 0) == b)
    pt = pt.astype(jnp.float32)
    acc_td[...] += _dot(pt, h2)
    acc_bu[...] += _dot(pt, h4)
    acc_cnt[...] += jnp.sum(pt, axis=1, keepdims=True)

    @pl.when(i == nblocks - 1)
    def _():
        inv = 1.0 / jnp.maximum(acc_cnt[...], 1.0)
        z = jnp.concatenate([acc_td[...] * inv, acc_bu[...] * inv], axis=1)
        logits = _dot(z, wfc_ref[...]) + bfc_ref[...]
        mx = jnp.max(logits, axis=1, keepdims=True)
        lse = jnp.log(jnp.sum(jnp.exp(logits - mx), axis=1, keepdims=True))
        out_ref[...] = (logits - mx) - lse


def kernel(x, edge_index, bu_edge_index, batch,
           W1, b1, W2, b2, W3, b3, W4, b4, Wfc, bfc):
    N0, D = x.shape
    H = W1.shape[1]
    C = Wfc.shape[1]
    E0 = edge_index.shape[1]
    f32 = jnp.float32

    N = ((N0 + 127) // 128) * 128
    if N != N0:
        x = jnp.concatenate([x, jnp.zeros((N - N0, D), f32)], axis=0)
        batch = jnp.concatenate(
            [batch, jnp.full((N - N0,), _G, batch.dtype)], axis=0)

    EPQ = _NS * _K * 8
    E = ((E0 + EPQ - 1) // EPQ) * EPQ
    assert N > N0 or E == E0

    def prep(ei):
        src, dst = ei[0], ei[1]
        if E != E0:
            pad = jnp.full((E - E0,), N0, jnp.int32)
            src = jnp.concatenate([src, pad])
            dst = jnp.concatenate([dst, pad])
        return src.reshape(E // _K, _K), dst.reshape(E // _K, _K)

    src_td, dst_td = prep(edge_index)
    src_bu, dst_bu = prep(bu_edge_index)
    dst2d_td, dst2d_bu = dst_td, dst_bu

    zeros_n = jnp.zeros((N // _NS,), f32)
    ones_k = jnp.ones((_K,), f32)
    zeros_nd = jnp.zeros((N // _NS, D), f32)

    deg_td, deg_bu = _make_deg_kernel(E, N)(dst2d_td, dst2d_bu,
                                            zeros_n, ones_k)
    deg_td = deg_td.reshape(N, 1)
    deg_bu = deg_bu.reshape(N, 1)

    NB = 8
    R = N // NB
    row = pl.BlockSpec((R, H), lambda i: (i, 0))
    row1 = pl.BlockSpec((R, 1), lambda i: (i, 0))
    wspec = pl.BlockSpec((H, H), lambda i: (0, 0))
    bspec = pl.BlockSpec((1, H), lambda i: (0, 0))

    y1, y3, itd, ibu = pl.pallas_call(
        _tc1_body,
        grid=(NB,),
        in_specs=[row, row1, row1, wspec, wspec],
        out_specs=[row, row, row1, row1],
        out_shape=[
            jax.ShapeDtypeStruct((N, H), f32),
            jax.ShapeDtypeStruct((N, H), f32),
            jax.ShapeDtypeStruct((N, 1), f32),
            jax.ShapeDtypeStruct((N, 1), f32),
        ],
    )(x, deg_td, deg_bu, W1, W3)

    agg = _make_agg_kernel(E, N, H)
    z1, z3 = agg(y1, src_td, dst_td, y3, src_bu, dst_bu, zeros_nd)

    y2, y4 = pl.pallas_call(
        _tc2_body,
        grid=(NB,),
        in_specs=[row, row, row1, bspec, wspec,
                  row, row, row1, bspec, wspec],
        out_specs=[row, row],
        out_shape=[
            jax.ShapeDtypeStruct((N, H), f32),
            jax.ShapeDtypeStruct((N, H), f32),
        ],
    )(z1, y1, itd, b1.reshape(1, H), W2,
      z3, y3, ibu, b3.reshape(1, H), W4)

    z2, z4 = agg(y2, src_td, dst_td, y4, src_bu, dst_bu, zeros_nd)

    out = pl.pallas_call(
        functools.partial(_tc3_body, NB),
        grid=(NB,),
        in_specs=[row, row, row1, bspec,
                  row, row, row1, bspec,
                  pl.BlockSpec((1, 1, R), lambda i: (i, 0, 0)),
                  pl.BlockSpec((2 * H, C), lambda i: (0, 0)),
                  pl.BlockSpec((1, C), lambda i: (0, 0))],
        out_specs=pl.BlockSpec((_G, C), lambda i: (0, 0)),
        out_shape=jax.ShapeDtypeStruct((_G, C), f32),
        scratch_shapes=[
            pltpu.VMEM((_G, H), f32),
            pltpu.VMEM((_G, H), f32),
            pltpu.VMEM((_G, 1), f32),
        ],
    )(z2, y2, itd, b2.reshape(1, H),
      z4, y4, ibu, b4.reshape(1, H),
      batch.reshape(NB, 1, R), Wfc, bfc.reshape(1, C))

    return out


# speedup vs baseline: 13.8640x; 1.0937x over previous
import functools

import jax
import jax.numpy as jnp
from jax import lax
from jax.experimental import pallas as pl
from jax.experimental.pallas import tpu as pltpu
from jax.experimental.pallas import tpu_sc as plsc

_NC = 2
_NS = 16
_K = 128
_G = 64


def _sc_mesh():
    return plsc.VectorSubcoreMesh(core_axis_name="c", subcore_axis_name="s")


@functools.lru_cache(maxsize=None)
def _make_deg_kernel(EP, N):
    assert EP % (_NS * _K) == 0 and N % _NS == 0
    chunks = EP // (_NS * _K)
    assert chunks % 8 == 0
    rows_per_tile = N // _NS
    _FK = 8

    @functools.partial(
        pl.kernel,
        out_type=[jax.ShapeDtypeStruct((N,), jnp.float32)] * 2,
        mesh=_sc_mesh(),
        scratch_types=[
            pltpu.VMEM((chunks, _K), jnp.int32),
            pltpu.VMEM((_K,), jnp.float32),
            pltpu.VMEM((N // _NS,), jnp.float32),
            pltpu.VMEM_SHARED((N,), jnp.float32),
            pltpu.SemaphoreType.DMA,
        ],
    )
    def deg_kernel(dst_td, dst_bu, zeros_n, ones_k, out_td, out_bu,
                   didx_v, ones_v, stage_v, hist_s, sem):
        c = lax.axis_index("c")
        s = lax.axis_index("s")
        r0 = s * rows_per_tile
        pltpu.sync_copy(zeros_n, stage_v)
        pltpu.sync_copy(stage_v, hist_s.at[pl.ds(r0, rows_per_tile)])
        pltpu.sync_copy(ones_k, ones_v)
        plsc.subcore_barrier()

        def run(dst2d):
            pltpu.sync_copy(dst2d.at[pl.ds(s * chunks, chunks)], didx_v)

            def group(g, carry):
                for q in range(_FK):
                    pltpu.async_copy(
                        ones_v, hist_s.at[didx_v.at[g * _FK + q]], sem,
                        add=True)
                for q in range(_FK):
                    pltpu.make_async_copy(
                        ones_v, hist_s.at[didx_v.at[g * _FK + q]], sem).wait()
                return carry

            lax.fori_loop(0, chunks // _FK, group, 0)

        @pl.when(c == 0)
        def _():
            run(dst_td)

        @pl.when(c == 1)
        def _():
            run(dst_bu)

        plsc.subcore_barrier()

        pltpu.sync_copy(hist_s.at[pl.ds(r0, rows_per_tile)], stage_v)

        @pl.when(c == 0)
        def _():
            pltpu.sync_copy(stage_v, out_td.at[pl.ds(r0, rows_per_tile)])

        @pl.when(c == 1)
        def _():
            pltpu.sync_copy(stage_v, out_bu.at[pl.ds(r0, rows_per_tile)])

    return deg_kernel


@functools.lru_cache(maxsize=None)
def _make_agg_kernel(EP, N, D):
    assert EP % (_NS * _K) == 0 and N % _NS == 0
    chunks = EP // (_NS * _K)
    assert chunks % 3 == 0
    rows_per_tile = N // _NS

    @functools.partial(
        pl.kernel,
        out_type=[jax.ShapeDtypeStruct((N, D), jnp.float32)] * 2,
        mesh=_sc_mesh(),
        scratch_types=[
            pltpu.VMEM((_K,), jnp.int32),
            pltpu.VMEM((_K,), jnp.int32),
            pltpu.VMEM((_K,), jnp.int32),
            pltpu.VMEM((_K,), jnp.int32),
            pltpu.VMEM((_K,), jnp.int32),
            pltpu.VMEM((_K,), jnp.int32),
            pltpu.VMEM((_K, D), jnp.float32),
            pltpu.VMEM((_K, D), jnp.float32),
            pltpu.VMEM((_K, D), jnp.float32),
            pltpu.VMEM_SHARED((N, D), jnp.float32),
            pltpu.SemaphoreType.DMA,
            pltpu.SemaphoreType.DMA,
            pltpu.SemaphoreType.DMA,
            pltpu.SemaphoreType.DMA,
            pltpu.SemaphoreType.DMA,
            pltpu.SemaphoreType.DMA,
        ],
    )
    def agg_kernel(y_td, src_td, dst_td, y_bu, src_bu, dst_bu, zeros_nd,
                   z_td, z_bu, sidx0, sidx1, sidx2, didx0, didx1, didx2,
                   buf0, buf1, buf2, acc_s,
                   gsem0, gsem1, gsem2, ssem0, ssem1, ssem2):
        c = lax.axis_index("c")
        s = lax.axis_index("s")
        r0 = s * rows_per_tile
        pltpu.sync_copy(zeros_nd, acc_s.at[pl.ds(r0, rows_per_tile)])
        plsc.subcore_barrier()

        sidx = (sidx0, sidx1, sidx2)
        didx = (didx0, didx1, didx2)
        bufs = (buf0, buf1, buf2)
        gsem = (gsem0, gsem1, gsem2)
        ssem = (ssem0, ssem1, ssem2)

        def run(y_hbm, src_hbm, dst_hbm):
            base0 = s * chunks * _K

            def iload(i, r):
                b = pl.multiple_of(base0 + i * _K, 8)
                pltpu.sync_copy(src_hbm.at[pl.ds(b, _K)], sidx[r])
                pltpu.sync_copy(dst_hbm.at[pl.ds(b, _K)], didx[r])

            def gstart(r):
                pltpu.async_copy(y_hbm.at[sidx[r]], bufs[r], gsem[r])

            def gwait(r):
                pltpu.make_async_copy(y_hbm.at[sidx[r]], bufs[r],
                                      gsem[r]).wait()

            def sstart(r):
                pltpu.async_copy(bufs[r], acc_s.at[didx[r]], ssem[r],
                                 add=True)

            def swait(r):
                pltpu.make_async_copy(bufs[r], acc_s.at[didx[r]],
                                      ssem[r]).wait()

            iload(0, 0)
            gstart(0)
            iload(1, 1)
            gstart(1)

            def tri(g, carry):
                i0 = 3 * g
                for u in range(3):
                    i = i0 + u
                    rn = (u + 2) % 3
                    if u == 0:
                        @pl.when(i0 > 0)
                        def _():
                            swait(rn)
                    else:
                        swait(rn)

                    @pl.when(i + 2 < chunks)
                    def _():
                        iload(i + 2, rn)
                        gstart(rn)

                    gwait(u)
                    sstart(u)
                return carry

            lax.fori_loop(0, chunks // 3, tri, 0)
            swait((chunks - 1) % 3)

        @pl.when(c == 0)
        def _():
            run(y_td, src_td, dst_td)

        @pl.when(c == 1)
        def _():
            run(y_bu, src_bu, dst_bu)

        plsc.subcore_barrier()

        @pl.when(c == 0)
        def _():
            pltpu.sync_copy(acc_s.at[pl.ds(r0, rows_per_tile)],
                            z_td.at[pl.ds(r0, rows_per_tile)])

        @pl.when(c == 1)
        def _():
            pltpu.sync_copy(acc_s.at[pl.ds(r0, rows_per_tile)],
                            z_bu.at[pl.ds(r0, rows_per_tile)])

    return agg_kernel


def _dot(a, b):
    return jnp.dot(a, b, preferred_element_type=jnp.float32,
                   precision=lax.Precision.HIGHEST)


def _elu(v):
    return jnp.where(v > 0.0, v, jnp.exp(jnp.minimum(v, 0.0)) - 1.0)


def _tc1_body(x_ref, degtd_ref, degbu_ref, w1_ref, w3_ref,
              y1_ref, y3_ref, itd_ref, ibu_ref):
    itd = lax.rsqrt(degtd_ref[...] + 1.0)
    ibu = lax.rsqrt(degbu_ref[...] + 1.0)
    xv = x_ref[...]
    y1_ref[...] = itd * _dot(xv, w1_ref[...])
    y3_ref[...] = ibu * _dot(xv, w3_ref[...])
    itd_ref[...] = itd
    ibu_ref[...] = ibu


def _tc2_body(z1_ref, y1_ref, itd_ref, b1_ref, w2_ref,
              z3_ref, y3_ref, ibu_ref, b3_ref, w4_ref,
              y2_ref, y4_ref):
    itd = itd_ref[...]
    ibu = ibu_ref[...]
    h1 = _elu(itd * (z1_ref[...] + y1_ref[...]) + b1_ref[...])
    h3 = _elu(ibu * (z3_ref[...] + y3_ref[...]) + b3_ref[...])
    y2_ref[...] = itd * _dot(h1, w2_ref[...])
    y4_ref[...] = ibu * _dot(h3, w4_ref[...])


def _tc3_body(nblocks, z2_ref, y2_ref, itd_ref, b2_ref,
              z4_ref, y4_ref, ibu_ref, b4_ref,
              batch_ref, wfc_ref, bfc_ref, out_ref,
              acc_td, acc_bu, acc_cnt):
    i = pl.program_id(0)

    @pl.when(i == 0)
    def _():
        acc_td[...] = jnp.zeros_like(acc_td)
        acc_bu[...] = jnp.zeros_like(acc_bu)
        acc_cnt[...] = jnp.zeros_like(acc_cnt)

    h2 = _elu(itd_ref[...] * (z2_ref[...] + y2_ref[...]) + b2_ref[...])
    h4 = _elu(ibu_ref[...] * (z4_ref[...] + y4_ref[...]) + b4_ref[...])
    r = h2.shape[0]
    b = batch_ref[0]
    pt = (lax.broadcasted_iota(jnp.int32, (_G, r), 0) == b)
    pt = pt.astype(jnp.float32)
    acc_td[...] += _dot(pt, h2)
    acc_bu[...] += _dot(pt, h4)
    acc_cnt[...] += jnp.sum(pt, axis=1, keepdims=True)

    @pl.when(i == nblocks - 1)
    def _():
        inv = 1.0 / jnp.maximum(acc_cnt[...], 1.0)
        z = jnp.concatenate([acc_td[...] * inv, acc_bu[...] * inv], axis=1)
        logits = _dot(z, wfc_ref[...]) + bfc_ref[...]
        mx = jnp.max(logits, axis=1, keepdims=True)
        lse = jnp.log(jnp.sum(jnp.exp(logits - mx), axis=1, keepdims=True))
        out_ref[...] = (logits - mx) - lse


def kernel(x, edge_index, bu_edge_index, batch,
           W1, b1, W2, b2, W3, b3, W4, b4, Wfc, bfc):
    N0, D = x.shape
    H = W1.shape[1]
    C = Wfc.shape[1]
    E0 = edge_index.shape[1]
    f32 = jnp.float32

    N = ((N0 + 127) // 128) * 128
    if N != N0:
        x = jnp.concatenate([x, jnp.zeros((N - N0, D), f32)], axis=0)
        batch = jnp.concatenate(
            [batch, jnp.full((N - N0,), _G, batch.dtype)], axis=0)

    EQA = _NS * _K * 3
    EA = ((E0 + EQA - 1) // EQA) * EQA
    EQD = _NS * _K * 8
    ED = ((E0 + EQD - 1) // EQD) * EQD
    assert N > N0 or (EA == E0 and ED == E0)

    def pad_to(v, length):
        if length == E0:
            return v
        return jnp.concatenate(
            [v, jnp.full((length - E0,), N0, jnp.int32)])

    def prep(ei):
        src, dst = ei[0], ei[1]
        return (pad_to(src, EA), pad_to(dst, EA),
                pad_to(dst, ED).reshape(ED // _K, _K))

    src_td, dst_td, dst2d_td = prep(edge_index)
    src_bu, dst_bu, dst2d_bu = prep(bu_edge_index)
    E = EA

    zeros_n = jnp.zeros((N // _NS,), f32)
    ones_k = jnp.ones((_K,), f32)
    zeros_nd = jnp.zeros((N // _NS, D), f32)

    deg_td, deg_bu = _make_deg_kernel(ED, N)(dst2d_td, dst2d_bu,
                                             zeros_n, ones_k)
    deg_td = deg_td.reshape(N, 1)
    deg_bu = deg_bu.reshape(N, 1)

    NB = 8
    R = N // NB
    row = pl.BlockSpec((R, H), lambda i: (i, 0))
    row1 = pl.BlockSpec((R, 1), lambda i: (i, 0))
    wspec = pl.BlockSpec((H, H), lambda i: (0, 0))
    bspec = pl.BlockSpec((1, H), lambda i: (0, 0))

    y1, y3, itd, ibu = pl.pallas_call(
        _tc1_body,
        grid=(NB,),
        in_specs=[row, row1, row1, wspec, wspec],
        out_specs=[row, row, row1, row1],
        out_shape=[
            jax.ShapeDtypeStruct((N, H), f32),
            jax.ShapeDtypeStruct((N, H), f32),
            jax.ShapeDtypeStruct((N, 1), f32),
            jax.ShapeDtypeStruct((N, 1), f32),
        ],
    )(x, deg_td, deg_bu, W1, W3)

    agg = _make_agg_kernel(E, N, H)
    z1, z3 = agg(y1, src_td, dst_td, y3, src_bu, dst_bu, zeros_nd)

    y2, y4 = pl.pallas_call(
        _tc2_body,
        grid=(NB,),
        in_specs=[row, row, row1, bspec, wspec,
                  row, row, row1, bspec, wspec],
        out_specs=[row, row],
        out_shape=[
            jax.ShapeDtypeStruct((N, H), f32),
            jax.ShapeDtypeStruct((N, H), f32),
        ],
    )(z1, y1, itd, b1.reshape(1, H), W2,
      z3, y3, ibu, b3.reshape(1, H), W4)

    z2, z4 = agg(y2, src_td, dst_td, y4, src_bu, dst_bu, zeros_nd)

    out = pl.pallas_call(
        functools.partial(_tc3_body, NB),
        grid=(NB,),
        in_specs=[row, row, row1, bspec,
                  row, row, row1, bspec,
                  pl.BlockSpec((1, 1, R), lambda i: (i, 0, 0)),
                  pl.BlockSpec((2 * H, C), lambda i: (0, 0)),
                  pl.BlockSpec((1, C), lambda i: (0, 0))],
        out_specs=pl.BlockSpec((_G, C), lambda i: (0, 0)),
        out_shape=jax.ShapeDtypeStruct((_G, C), f32),
        scratch_shapes=[
            pltpu.VMEM((_G, H), f32),
            pltpu.VMEM((_G, H), f32),
            pltpu.VMEM((_G, 1), f32),
        ],
    )(z2, y2, itd, b2.reshape(1, H),
      z4, y4, ibu, b4.reshape(1, H),
      batch.reshape(NB, 1, R), Wfc, bfc.reshape(1, C))

    return out


# speedup vs baseline: 23.3014x; 1.6807x over previous
import functools

import jax
import jax.numpy as jnp
from jax import lax
from jax.experimental import pallas as pl
from jax.experimental.pallas import tpu as pltpu
from jax.experimental.pallas import tpu_sc as plsc

_NC = 2
_NS = 16
_K = 128
_KA = 120
_G = 64


def _sc_mesh():
    return plsc.VectorSubcoreMesh(core_axis_name="c", subcore_axis_name="s")


@functools.lru_cache(maxsize=None)
def _make_deg_kernel(EP, N):
    assert EP % (_NS * _K) == 0 and N % _NS == 0
    chunks = EP // (_NS * _K)
    assert chunks % 8 == 0
    rows_per_tile = N // _NS
    _FK = 8

    @functools.partial(
        pl.kernel,
        out_type=[jax.ShapeDtypeStruct((N,), jnp.float32)] * 2,
        mesh=_sc_mesh(),
        scratch_types=[
            pltpu.VMEM((chunks, _K), jnp.int32),
            pltpu.VMEM((_K,), jnp.float32),
            pltpu.VMEM((N // _NS,), jnp.float32),
            pltpu.VMEM_SHARED((N,), jnp.float32),
            pltpu.SemaphoreType.DMA,
        ],
    )
    def deg_kernel(dst_td, dst_bu, zeros_n, ones_k, out_td, out_bu,
                   didx_v, ones_v, stage_v, hist_s, sem):
        c = lax.axis_index("c")
        s = lax.axis_index("s")
        r0 = s * rows_per_tile
        pltpu.sync_copy(zeros_n, stage_v)
        pltpu.sync_copy(stage_v, hist_s.at[pl.ds(r0, rows_per_tile)])
        pltpu.sync_copy(ones_k, ones_v)
        plsc.subcore_barrier()

        def run(dst2d):
            pltpu.sync_copy(dst2d.at[pl.ds(s * chunks, chunks)], didx_v)

            def group(g, carry):
                for q in range(_FK):
                    pltpu.async_copy(
                        ones_v, hist_s.at[didx_v.at[g * _FK + q]], sem,
                        add=True)
                for q in range(_FK):
                    pltpu.make_async_copy(
                        ones_v, hist_s.at[didx_v.at[g * _FK + q]], sem).wait()
                return carry

            lax.fori_loop(0, chunks // _FK, group, 0)

        @pl.when(c == 0)
        def _():
            run(dst_td)

        @pl.when(c == 1)
        def _():
            run(dst_bu)

        plsc.subcore_barrier()

        pltpu.sync_copy(hist_s.at[pl.ds(r0, rows_per_tile)], stage_v)

        @pl.when(c == 0)
        def _():
            pltpu.sync_copy(stage_v, out_td.at[pl.ds(r0, rows_per_tile)])

        @pl.when(c == 1)
        def _():
            pltpu.sync_copy(stage_v, out_bu.at[pl.ds(r0, rows_per_tile)])

    return deg_kernel


@functools.lru_cache(maxsize=None)
def _make_agg_kernel(EP, N, D):
    KA = _KA
    assert EP % (_NS * KA) == 0 and N % _NS == 0
    chunks = EP // (_NS * KA)
    assert chunks % 6 == 0
    rows_per_tile = N // _NS

    @functools.partial(
        pl.kernel,
        out_type=[jax.ShapeDtypeStruct((N, D), jnp.float32)] * 2,
        mesh=_sc_mesh(),
        scratch_types=(
            [pltpu.VMEM((KA,), jnp.int32)] * 6
            + [pltpu.VMEM((KA,), jnp.int32)] * 6
            + [pltpu.VMEM((KA, D), jnp.float32)] * 3
            + [pltpu.VMEM_SHARED((N, D), jnp.float32)]
            + [pltpu.SemaphoreType.DMA] * 12
        ),
    )
    def agg_kernel(y_td, src_td, dst_td, y_bu, src_bu, dst_bu, zeros_nd,
                   z_td, z_bu,
                   si0, si1, si2, si3, si4, si5,
                   di0, di1, di2, di3, di4, di5,
                   buf0, buf1, buf2, acc_s,
                   gs0, gs1, gs2, ss0, ss1, ss2,
                   is0, is1, is2, is3, is4, is5):
        c = lax.axis_index("c")
        s = lax.axis_index("s")
        r0 = s * rows_per_tile
        pltpu.sync_copy(zeros_nd, acc_s.at[pl.ds(r0, rows_per_tile)])
        plsc.subcore_barrier()

        sidx = (si0, si1, si2, si3, si4, si5)
        didx = (di0, di1, di2, di3, di4, di5)
        bufs = (buf0, buf1, buf2)
        gsem = (gs0, gs1, gs2)
        ssem = (ss0, ss1, ss2)
        isem = (is0, is1, is2, is3, is4, is5)

        def run(y_hbm, src_hbm, dst_hbm):
            base0 = s * chunks * KA

            def iload(i, p):
                b = pl.multiple_of(base0 + i * KA, 8)
                pltpu.async_copy(src_hbm.at[pl.ds(b, KA)], sidx[p], isem[p])
                pltpu.async_copy(dst_hbm.at[pl.ds(b, KA)], didx[p], isem[p])

            def iwait(p):
                pltpu.make_async_copy(src_hbm.at[pl.ds(base0, KA)],
                                      sidx[p], isem[p]).wait()
                pltpu.make_async_copy(dst_hbm.at[pl.ds(base0, KA)],
                                      didx[p], isem[p]).wait()

            def gstart(r, p):
                pltpu.async_copy(y_hbm.at[sidx[p]], bufs[r], gsem[r])

            def gwait(r):
                pltpu.make_async_copy(y_hbm.at[sidx[0]], bufs[r],
                                      gsem[r]).wait()

            def sstart(r, p):
                pltpu.async_copy(bufs[r], acc_s.at[didx[p]], ssem[r],
                                 add=True)

            def swait(r):
                pltpu.make_async_copy(bufs[r], acc_s.at[didx[0]],
                                      ssem[r]).wait()

            for j in range(4):
                iload(j, j)
            iwait(0)
            gstart(0, 0)
            iwait(1)
            gstart(1, 1)

            def hexa(g, carry):
                i0 = 6 * g
                for u in range(6):
                    i = i0 + u
                    r, rn = u % 3, (u + 2) % 3
                    p, p2, p4 = u, (u + 2) % 6, (u + 4) % 6
                    if u == 0:
                        @pl.when(i0 > 0)
                        def _():
                            swait(rn)
                    else:
                        swait(rn)

                    @pl.when(i + 4 < chunks)
                    def _():
                        iload(i + 4, p4)

                    @pl.when(i + 2 < chunks)
                    def _():
                        iwait(p2)
                        gstart(rn, p2)

                    gwait(r)
                    sstart(r, p)
                return carry

            lax.fori_loop(0, chunks // 6, hexa, 0)
            swait((chunks - 1) % 3)

        @pl.when(c == 0)
        def _():
            run(y_td, src_td, dst_td)

        @pl.when(c == 1)
        def _():
            run(y_bu, src_bu, dst_bu)

        plsc.subcore_barrier()

        @pl.when(c == 0)
        def _():
            pltpu.sync_copy(acc_s.at[pl.ds(r0, rows_per_tile)],
                            z_td.at[pl.ds(r0, rows_per_tile)])

        @pl.when(c == 1)
        def _():
            pltpu.sync_copy(acc_s.at[pl.ds(r0, rows_per_tile)],
                            z_bu.at[pl.ds(r0, rows_per_tile)])

    return agg_kernel


def _dot(a, b):
    return jnp.dot(a, b, preferred_element_type=jnp.float32,
                   precision=lax.Precision.HIGHEST)


def _elu(v):
    return jnp.where(v > 0.0, v, jnp.exp(jnp.minimum(v, 0.0)) - 1.0)


def _tc1_body(x_ref, degtd_ref, degbu_ref, w1_ref, w3_ref,
              y1_ref, y3_ref, itd_ref, ibu_ref):
    itd = lax.rsqrt(degtd_ref[...] + 1.0)
    ibu = lax.rsqrt(degbu_ref[...] + 1.0)
    xv = x_ref[...]
    y1_ref[...] = itd * _dot(xv, w1_ref[...])
    y3_ref[...] = ibu * _dot(xv, w3_ref[...])
    itd_ref[...] = itd
    ibu_ref[...] = ibu


def _tc2_body(z1_ref, y1_ref, itd_ref, b1_ref, w2_ref,
              z3_ref, y3_ref, ibu_ref, b3_ref, w4_ref,
              y2_ref, y4_ref):
    itd = itd_ref[...]
    ibu = ibu_ref[...]
    h1 = _elu(itd * (z1_ref[...] + y1_ref[...]) + b1_ref[...])
    h3 = _elu(ibu * (z3_ref[...] + y3_ref[...]) + b3_ref[...])
    y2_ref[...] = itd * _dot(h1, w2_ref[...])
    y4_ref[...] = ibu * _dot(h3, w4_ref[...])


def _tc3_body(nblocks, z2_ref, y2_ref, itd_ref, b2_ref,
              z4_ref, y4_ref, ibu_ref, b4_ref,
              batch_ref, wfc_ref, bfc_ref, out_ref,
              acc_td, acc_bu, acc_cnt):
    i = pl.program_id(0)

    @pl.when(i == 0)
    def _():
        acc_td[...] = jnp.zeros_like(acc_td)
        acc_bu[...] = jnp.zeros_like(acc_bu)
        acc_cnt[...] = jnp.zeros_like(acc_cnt)

    h2 = _elu(itd_ref[...] * (z2_ref[...] + y2_ref[...]) + b2_ref[...])
    h4 = _elu(ibu_ref[...] * (z4_ref[...] + y4_ref[...]) + b4_ref[...])
    r = h2.shape[0]
    b = batch_ref[0]
    pt = (lax.broadcasted_iota(jnp.int32, (_G, r), 0) == b)
    pt = pt.astype(jnp.float32)
    acc_td[...] += _dot(pt, h2)
    acc_bu[...] += _dot(pt, h4)
    acc_cnt[...] += jnp.sum(pt, axis=1, keepdims=True)

    @pl.when(i == nblocks - 1)
    def _():
        inv = 1.0 / jnp.maximum(acc_cnt[...], 1.0)
        z = jnp.concatenate([acc_td[...] * inv, acc_bu[...] * inv], axis=1)
        logits = _dot(z, wfc_ref[...]) + bfc_ref[...]
        mx = jnp.max(logits, axis=1, keepdims=True)
        lse = jnp.log(jnp.sum(jnp.exp(logits - mx), axis=1, keepdims=True))
        out_ref[...] = (logits - mx) - lse


def kernel(x, edge_index, bu_edge_index, batch,
           W1, b1, W2, b2, W3, b3, W4, b4, Wfc, bfc):
    N0, D = x.shape
    H = W1.shape[1]
    C = Wfc.shape[1]
    E0 = edge_index.shape[1]
    f32 = jnp.float32

    N = ((N0 + 127) // 128) * 128
    if N != N0:
        x = jnp.concatenate([x, jnp.zeros((N - N0, D), f32)], axis=0)
        batch = jnp.concatenate(
            [batch, jnp.full((N - N0,), _G, batch.dtype)], axis=0)

    EQA = _NS * _KA * 6
    EA = ((E0 + EQA - 1) // EQA) * EQA
    EQD = _NS * _K * 8
    ED = ((E0 + EQD - 1) // EQD) * EQD
    assert N > N0 or (EA == E0 and ED == E0)

    def pad_to(v, length):
        if length == E0:
            return v
        return jnp.concatenate(
            [v, jnp.full((length - E0,), N0, jnp.int32)])

    def prep(ei):
        src, dst = ei[0], ei[1]
        return (pad_to(src, EA), pad_to(dst, EA),
                pad_to(dst, ED).reshape(ED // _K, _K))

    src_td, dst_td, dst2d_td = prep(edge_index)
    src_bu, dst_bu, dst2d_bu = prep(bu_edge_index)
    E = EA

    zeros_n = jnp.zeros((N // _NS,), f32)
    ones_k = jnp.ones((_K,), f32)
    zeros_nd = jnp.zeros((N // _NS, D), f32)

    deg_td, deg_bu = _make_deg_kernel(ED, N)(dst2d_td, dst2d_bu,
                                             zeros_n, ones_k)
    deg_td = deg_td.reshape(N, 1)
    deg_bu = deg_bu.reshape(N, 1)

    NB = 8
    R = N // NB
    row = pl.BlockSpec((R, H), lambda i: (i, 0))
    row1 = pl.BlockSpec((R, 1), lambda i: (i, 0))
    wspec = pl.BlockSpec((H, H), lambda i: (0, 0))
    bspec = pl.BlockSpec((1, H), lambda i: (0, 0))

    y1, y3, itd, ibu = pl.pallas_call(
        _tc1_body,
        grid=(NB,),
        in_specs=[row, row1, row1, wspec, wspec],
        out_specs=[row, row, row1, row1],
        out_shape=[
            jax.ShapeDtypeStruct((N, H), f32),
            jax.ShapeDtypeStruct((N, H), f32),
            jax.ShapeDtypeStruct((N, 1), f32),
            jax.ShapeDtypeStruct((N, 1), f32),
        ],
    )(x, deg_td, deg_bu, W1, W3)

    agg = _make_agg_kernel(E, N, H)
    z1, z3 = agg(y1, src_td, dst_td, y3, src_bu, dst_bu, zeros_nd)

    y2, y4 = pl.pallas_call(
        _tc2_body,
        grid=(NB,),
        in_specs=[row, row, row1, bspec, wspec,
                  row, row, row1, bspec, wspec],
        out_specs=[row, row],
        out_shape=[
            jax.ShapeDtypeStruct((N, H), f32),
            jax.ShapeDtypeStruct((N, H), f32),
        ],
    )(z1, y1, itd, b1.reshape(1, H), W2,
      z3, y3, ibu, b3.reshape(1, H), W4)

    z2, z4 = agg(y2, src_td, dst_td, y4, src_bu, dst_bu, zeros_nd)

    out = pl.pallas_call(
        functools.partial(_tc3_body, NB),
        grid=(NB,),
        in_specs=[row, row, row1, bspec,
                  row, row, row1, bspec,
                  pl.BlockSpec((1, 1, R), lambda i: (i, 0, 0)),
                  pl.BlockSpec((2 * H, C), lambda i: (0, 0)),
                  pl.BlockSpec((1, C), lambda i: (0, 0))],
        out_specs=pl.BlockSpec((_G, C), lambda i: (0, 0)),
        out_shape=jax.ShapeDtypeStruct((_G, C), f32),
        scratch_shapes=[
            pltpu.VMEM((_G, H), f32),
            pltpu.VMEM((_G, H), f32),
            pltpu.VMEM((_G, 1), f32),
        ],
    )(z2, y2, itd, b2.reshape(1, H),
      z4, y4, ibu, b4.reshape(1, H),
      batch.reshape(NB, 1, R), Wfc, bfc.reshape(1, C))

    return out
